# baseline (speedup 1.0000x reference)
import functools

import jax
import jax.numpy as jnp
from jax import lax
from jax.experimental import pallas as pl
from jax.experimental.pallas import tpu as pltpu
from jax.experimental.pallas import tpu_sc as plsc

N = 10000
NPAD = 10240
E = 160000
EPAD = 163840
NCLASS = 40
F = 256
FC = 128
NSUB = 16
NCORE = 2
EPS = EPAD // (NSUB * NCORE)
CH = 64
J = EPS // CH
J2 = EPAD // NSUB // CH
BT = 640
RT = NPAD // NSUB
NBUF = 3


def _mesh():
    return plsc.VectorSubcoreMesh(
        core_axis_name="c", subcore_axis_name="s",
        num_cores=NCORE, num_subcores=NSUB)


@functools.lru_cache()
def _make_spmm(nf):
    @functools.partial(
        pl.kernel,
        out_type=jax.ShapeDtypeStruct((NCORE, nf, NPAD, FC), jnp.float32),
        mesh=_mesh(),
        scratch_types=[
            pltpu.VMEM_SHARED((NPAD, FC), jnp.float32),
            pltpu.VMEM((J, CH), jnp.int32),
            pltpu.VMEM((J, CH), jnp.int32),
            [pltpu.VMEM((CH, FC), jnp.float32) for _ in range(NBUF)],
            [pltpu.SemaphoreType.DMA for _ in range(NBUF)],
        ],
    )
    def spmm(data, gidx, sidx, zrows, out, acc, gi, si, bufs, sems):
        c = lax.axis_index("c")
        s = lax.axis_index("s")
        rbase = s * RT
        @pl.when(c == 0)
        def _():
            pltpu.sync_copy(gidx.at[s], gi)
            pltpu.sync_copy(sidx.at[s], si)
        @pl.when(c == 1)
        def _():
            pltpu.sync_copy(gidx.at[NSUB + s], gi)
            pltpu.sync_copy(sidx.at[NSUB + s], si)
        for q in range(nf):
            plane = data.at[q]
            pltpu.sync_copy(zrows, acc.at[pl.ds(rbase, RT)])
            plsc.subcore_barrier()
            for b in range(NBUF - 1):
                pltpu.async_copy(plane.at[gi.at[b]], bufs[b], sems[b])
            def body(i, _):
                for b in range(NBUF):
                    j = i * NBUF + b
                    pltpu.make_async_copy(
                        plane.at[gi.at[j]], bufs[b], sems[b]).wait()
                    pltpu.sync_copy(bufs[b], acc.at[si.at[j]], add=True)
                    nb = (b + NBUF - 1) % NBUF
                    @pl.when(j + NBUF - 1 < J)
                    def _():
                        pltpu.async_copy(
                            plane.at[gi.at[j + NBUF - 1]], bufs[nb], sems[nb])
                return 0
            lax.fori_loop(0, J // NBUF, body, 0)
            for j in range(J - J % NBUF, J):
                b = j % NBUF
                pltpu.make_async_copy(
                    plane.at[gi.at[j]], bufs[b], sems[b]).wait()
                pltpu.sync_copy(bufs[b], acc.at[si.at[j]], add=True)
            plsc.subcore_barrier()
            @pl.when(c == 0)
            def _():
                pltpu.sync_copy(acc.at[pl.ds(rbase, RT)],
                                out.at[0, q, pl.ds(rbase, RT)])
            @pl.when(c == 1)
            def _():
                pltpu.sync_copy(acc.at[pl.ds(rbase, RT)],
                                out.at[1, q, pl.ds(rbase, RT)])
            if q + 1 < nf:
                plsc.subcore_barrier()

    return spmm


@functools.lru_cache()
def _make_counts():
    @functools.partial(
        pl.kernel,
        out_type=jax.ShapeDtypeStruct((NCORE, NPAD, FC), jnp.float32),
        mesh=_mesh(),
        scratch_types=[
            pltpu.VMEM_SHARED((NPAD, FC), jnp.float32),
            pltpu.VMEM((J2, CH), jnp.int32),
            pltpu.VMEM((CH, FC), jnp.float32),
        ],
    )
    def counts(eidx, zrows, ones, out, acc, ii, ob):
        c = lax.axis_index("c")
        s = lax.axis_index("s")
        rbase = s * RT
        @pl.when(c == 0)
        def _():
            pltpu.sync_copy(eidx.at[s], ii)
        @pl.when(c == 1)
        def _():
            pltpu.sync_copy(eidx.at[NSUB + s], ii)
        pltpu.sync_copy(ones, ob)
        pltpu.sync_copy(zrows, acc.at[pl.ds(rbase, RT)])
        plsc.subcore_barrier()
        def body(j, _):
            pltpu.sync_copy(ob, acc.at[ii.at[j]], add=True)
            return 0
        lax.fori_loop(0, J2, body, 0)
        plsc.subcore_barrier()
        @pl.when(c == 0)
        def _():
            pltpu.sync_copy(acc.at[pl.ds(rbase, RT)],
                            out.at[0, pl.ds(rbase, RT)])
        @pl.when(c == 1)
        def _():
            pltpu.sync_copy(acc.at[pl.ds(rbase, RT)],
                            out.at[1, pl.ds(rbase, RT)])

    return counts


def _dg(a, w):
    return lax.dot_general(a, w, (((1,), (1,)), ((), ())),
                           preferred_element_type=jnp.float32)


def _split(y):
    return jnp.concatenate([y[0], y[1]], axis=1)


_P2 = pl.BlockSpec((2, BT, FC), lambda i: (0, i, 0))
_S2 = pl.BlockSpec((2, 2, BT, FC), lambda i: (0, 0, i, 0))
_S4 = pl.BlockSpec((2, 4, BT, FC), lambda i: (0, 0, i, 0))
_R = pl.BlockSpec((BT, F), lambda i: (i, 0))
_RC = pl.BlockSpec((BT, FC), lambda i: (i, 0))
_W = pl.BlockSpec((F, F), lambda i: (0, 0))
_B = pl.BlockSpec((1, F), lambda i: (0, 0))


def _h3_struct():
    return jax.ShapeDtypeStruct((2, NPAD, FC), jnp.float32)


def _enc_body(x_ref, w_ref, b_ref, o_ref):
    y = jnp.maximum(_dg(x_ref[...], w_ref[...]) + b_ref[...], 0.0)
    o_ref[0] = y[:, :FC]
    o_ref[1] = y[:, FC:]


def _enc(x, w, b):
    return pl.pallas_call(
        _enc_body,
        grid=(NPAD // BT,),
        in_specs=[_R, _W, _B],
        out_specs=_P2,
        out_shape=_h3_struct(),
    )(x, w, b)


def _dec_body(h_ref, w_ref, b_ref, o_ref):
    o_ref[...] = _dg(_split(h_ref[...]), w_ref[...]) + b_ref[...]


def _dec(h3, w, b):
    return pl.pallas_call(
        _dec_body,
        grid=(NPAD // BT,),
        in_specs=[
            _P2,
            pl.BlockSpec((FC, F), lambda i: (0, 0)),
            pl.BlockSpec((1, FC), lambda i: (0, 0)),
        ],
        out_specs=_RC,
        out_shape=jax.ShapeDtypeStruct((NPAD, FC), jnp.float32),
    )(h3, w, b)


def _layer_a_body(h_ref, agg_ref, cnt_ref, wl_ref, bl_ref, wr_ref,
                  gwl_ref, gbl_ref, gwr_ref, xa_ref, y_ref):
    cnt = cnt_ref[:, 0:1]
    r = 1.0 / jnp.maximum(cnt, 1.0)
    aggp = agg_ref[0] + agg_ref[1]
    agg = _split(aggp) * r
    h = _split(h_ref[...])
    xa = jnp.maximum(_dg(agg, wl_ref[...]) + bl_ref[...]
                     + _dg(h, wr_ref[...]), 0.0)
    xx = jnp.maximum(_dg(agg, gwl_ref[...]) + gbl_ref[...]
                     + _dg(h, gwr_ref[...]), 0.0)
    xa_ref[...] = xa
    x2 = xx * xx
    y_ref[0] = xx[:, :FC]
    y_ref[1] = xx[:, FC:]
    y_ref[2] = x2[:, :FC]
    y_ref[3] = x2[:, FC:]


def _layer_a(h3, aggs, cnt, wl, bl, wr, gwl, gbl, gwr):
    return pl.pallas_call(
        _layer_a_body,
        grid=(NPAD // BT,),
        in_specs=[_P2, _S2, _RC, _W, _B, _W, _W, _B, _W],
        out_specs=[_R, pl.BlockSpec((4, BT, FC), lambda i: (0, i, 0))],
        out_shape=[
            jax.ShapeDtypeStruct((NPAD, F), jnp.float32),
            jax.ShapeDtypeStruct((4, NPAD, FC), jnp.float32),
        ],
    )(h3, aggs, cnt, wl, bl, wr, gwl, gbl, gwr)


def _layer_b_body(h_ref, xa_ref, y_ref, ab_ref, cnt_ref, o_ref):
    cnt = cnt_ref[:, 0:1]
    r = 1.0 / jnp.maximum(cnt, 1.0)
    d = jnp.minimum(cnt, 1.0)
    xx = _split(y_ref[...])
    ab = ab_ref[0] + ab_ref[1]
    a = jnp.concatenate([ab[0], ab[1]], axis=1) * r
    b = jnp.concatenate([ab[2], ab[3]], axis=1) * r
    tau = jnp.tanh((d * xx - 2.0 * a) * xx + b)
    h = _split(h_ref[...])
    o = h + tau * (xa_ref[...] - h)
    o_ref[0] = o[:, :FC]
    o_ref[1] = o[:, FC:]


def _layer_b(h3, xa, y3, ab, cnt):
    return pl.pallas_call(
        _layer_b_body,
        grid=(NPAD // BT,),
        in_specs=[_P2, _R, _P2, _S4, _RC],
        out_specs=_P2,
        out_shape=_h3_struct(),
    )(h3, xa, y3, ab, cnt)


def kernel(x, edge_index, enc_W, enc_b, dec_W, dec_b,
           conv_Wl, conv_bl, conv_Wr, gg_Wl, gg_bl, gg_Wr):
    f32 = jnp.float32
    src = edge_index[0]
    dst = edge_index[1]
    pad = jnp.full((EPAD - E,), N, jnp.int32)
    srcf = jnp.concatenate([src, pad])
    dstf = jnp.concatenate([dst, pad])
    srcp = srcf.reshape(NSUB * NCORE, J, CH)
    dstp = dstf.reshape(NSUB * NCORE, J, CH)
    srcc = srcf.reshape(NSUB, J2, CH)
    dstc = dstf.reshape(NSUB, J2, CH)
    zrows = jnp.zeros((RT, FC), f32)
    ones = jnp.ones((CH, FC), f32)
    xp = jnp.pad(x, ((0, NPAD - N), (0, 0)))

    cnts = _make_counts()(jnp.concatenate([dstc, srcc], axis=0), zrows, ones)
    cnt_dst = cnts[0]
    cnt_src = cnts[1]
    spmm2 = _make_spmm(2)
    spmm4 = _make_spmm(4)

    blr = conv_bl.reshape(1, F)
    gblr = gg_bl.reshape(1, F)
    h3 = _enc(xp, enc_W, enc_b.reshape(1, F))
    for _ in range(4):
        aggs = spmm2(h3, srcp, dstp, zrows)
        xa, y3 = _layer_a(h3, aggs, cnt_dst, conv_Wl, blr, conv_Wr,
                          gg_Wl, gblr, gg_Wr)
        ab = spmm4(y3, dstp, srcp, zrows)
        h3 = _layer_b(h3, xa, y3, ab, cnt_src)

    dw = jnp.pad(dec_W, ((0, FC - NCLASS), (0, 0)))
    db = jnp.pad(dec_b, (0, FC - NCLASS)).reshape(1, FC)
    out = _dec(h3, dw, db)
    return out[:N, :NCLASS]


# speedup vs baseline: 1.4962x; 1.4962x over previous
import functools

import jax
import jax.numpy as jnp
from jax import lax
from jax.experimental import pallas as pl
from jax.experimental.pallas import tpu as pltpu
from jax.experimental.pallas import tpu_sc as plsc

N = 10000
NPAD = 10240
E = 160000
EPAD = 163840
NCLASS = 40
F = 256
PW = 128
NSUB = 16
NCORE = 2
EPS = EPAD // NSUB
B = 128
J = EPS // B
CC = 64
JC = EPS // CC
BT = 640
RT = NPAD // NSUB


def _mesh():
    return plsc.VectorSubcoreMesh(
        core_axis_name="c", subcore_axis_name="s",
        num_cores=NCORE, num_subcores=NSUB)


@functools.lru_cache()
def _make_spmm(npl):
    @functools.partial(
        pl.kernel,
        out_type=jax.ShapeDtypeStruct((npl, NPAD, PW), jnp.float32),
        mesh=_mesh(),
        scratch_types=[
            pltpu.VMEM_SHARED((NPAD, PW), jnp.float32),
            [pltpu.VMEM((B,), jnp.int32) for _ in range(2)],
            [pltpu.VMEM((B,), jnp.int32) for _ in range(2)],
            [pltpu.VMEM((B, PW), jnp.float32) for _ in range(2)],
            [pltpu.SemaphoreType.DMA for _ in range(2)],
            [pltpu.SemaphoreType.DMA for _ in range(2)],
        ],
    )
    def spmm(data, gidx, sidx, zrows, out, acc, gib, sib, dbuf, isems, gsems):
        c = lax.axis_index("c")
        s = lax.axis_index("s")
        rbase = s * RT
        for p in range(npl // 2):
            q = c + 2 * p
            plane = data.at[q]
            pltpu.sync_copy(zrows, acc.at[pl.ds(rbase, RT)])
            plsc.subcore_barrier()
            pltpu.sync_copy(gidx.at[s, 0], gib[0])
            pltpu.sync_copy(sidx.at[s, 0], sib[0])
            pltpu.async_copy(gidx.at[s, 1], gib[1], isems[1])
            pltpu.async_copy(sidx.at[s, 1], sib[1], isems[1])
            pltpu.async_copy(plane.at[gib[0]], dbuf[0], gsems[0])
            def body(i, _):
                for b in range(2):
                    j = 2 * i + b
                    b1 = 1 - b
                    @pl.when(j + 1 < J)
                    def _():
                        pltpu.make_async_copy(
                            gidx.at[s, j + 1], gib[b1], isems[b1]).wait()
                        pltpu.make_async_copy(
                            sidx.at[s, j + 1], sib[b1], isems[b1]).wait()
                        pltpu.async_copy(
                            plane.at[gib[b1]], dbuf[b1], gsems[b1])
                    pltpu.make_async_copy(
                        plane.at[gib[b]], dbuf[b], gsems[b]).wait()
                    pltpu.sync_copy(dbuf[b], acc.at[sib[b]], add=True)
                    @pl.when(j + 2 < J)
                    def _():
                        pltpu.async_copy(gidx.at[s, j + 2], gib[b], isems[b])
                        pltpu.async_copy(sidx.at[s, j + 2], sib[b], isems[b])
                return 0
            lax.fori_loop(0, J // 2, body, 0)
            plsc.subcore_barrier()
            @pl.when(c == 0)
            def _():
                pltpu.sync_copy(acc.at[pl.ds(rbase, RT)],
                                out.at[2 * p, pl.ds(rbase, RT)])
            @pl.when(c == 1)
            def _():
                pltpu.sync_copy(acc.at[pl.ds(rbase, RT)],
                                out.at[2 * p + 1, pl.ds(rbase, RT)])
            if p + 1 < npl // 2:
                plsc.subcore_barrier()

    return spmm


@functools.lru_cache()
def _make_counts():
    @functools.partial(
        pl.kernel,
        out_type=jax.ShapeDtypeStruct((NCORE, NPAD, 128), jnp.float32),
        mesh=_mesh(),
        scratch_types=[
            pltpu.VMEM_SHARED((NPAD, 128), jnp.float32),
            pltpu.VMEM((JC, CC), jnp.int32),
            pltpu.VMEM((CC, 128), jnp.float32),
        ],
    )
    def counts(eidx, zcnt, ones, out, acc, ii, ob):
        c = lax.axis_index("c")
        s = lax.axis_index("s")
        rbase = s * RT
        @pl.when(c == 0)
        def _():
            pltpu.sync_copy(eidx.at[s], ii)
        @pl.when(c == 1)
        def _():
            pltpu.sync_copy(eidx.at[NSUB + s], ii)
        pltpu.sync_copy(ones, ob)
        pltpu.sync_copy(zcnt, acc.at[pl.ds(rbase, RT)])
        plsc.subcore_barrier()
        def body(j, _):
            pltpu.sync_copy(ob, acc.at[ii.at[j]], add=True)
            return 0
        lax.fori_loop(0, JC, body, 0)
        plsc.subcore_barrier()
        @pl.when(c == 0)
        def _():
            pltpu.sync_copy(acc.at[pl.ds(rbase, RT)],
                            out.at[0, pl.ds(rbase, RT)])
        @pl.when(c == 1)
        def _():
            pltpu.sync_copy(acc.at[pl.ds(rbase, RT)],
                            out.at[1, pl.ds(rbase, RT)])

    return counts


def _dg(a, w):
    return lax.dot_general(a, w, (((1,), (1,)), ((), ())),
                           preferred_element_type=jnp.float32)


def _cat(pref):
    return jnp.concatenate([pref[i] for i in range(pref.shape[0])], axis=1)


def _planes(o_ref, y, npl):
    for qq in range(npl):
        o_ref[qq] = y[:, qq * PW:(qq + 1) * PW]


_P2 = pl.BlockSpec((2, BT, PW), lambda i: (0, i, 0))
_P4 = pl.BlockSpec((4, BT, PW), lambda i: (0, i, 0))
_R = pl.BlockSpec((BT, F), lambda i: (i, 0))
_RC = pl.BlockSpec((BT, 128), lambda i: (i, 0))
_W = pl.BlockSpec((F, F), lambda i: (0, 0))
_B = pl.BlockSpec((1, F), lambda i: (0, 0))


def _h_struct():
    return jax.ShapeDtypeStruct((2, NPAD, PW), jnp.float32)


def _enc_body(x_ref, w_ref, b_ref, o_ref):
    y = jnp.maximum(_dg(x_ref[...], w_ref[...]) + b_ref[...], 0.0)
    _planes(o_ref, y, 2)


def _enc(x, w, b):
    return pl.pallas_call(
        _enc_body,
        grid=(NPAD // BT,),
        in_specs=[_R, _W, _B],
        out_specs=_P2,
        out_shape=_h_struct(),
    )(x, w, b)


def _dec_body(h_ref, w_ref, b_ref, o_ref):
    o_ref[...] = _dg(_cat(h_ref[...]), w_ref[...]) + b_ref[...]


def _dec(h4, w, b):
    return pl.pallas_call(
        _dec_body,
        grid=(NPAD // BT,),
        in_specs=[
            _P2,
            pl.BlockSpec((128, F), lambda i: (0, 0)),
            pl.BlockSpec((1, 128), lambda i: (0, 0)),
        ],
        out_specs=_RC,
        out_shape=jax.ShapeDtypeStruct((NPAD, 128), jnp.float32),
    )(h4, w, b)


def _layer_a_body(h_ref, agg_ref, cnt_ref, wl_ref, bl_ref, wr_ref,
                  gwl_ref, gbl_ref, gwr_ref, xa_ref, y_ref):
    cnt = cnt_ref[:, 0:1]
    r = 1.0 / jnp.maximum(cnt, 1.0)
    agg = _cat(agg_ref[...]) * r
    h = _cat(h_ref[...])
    xa = jnp.maximum(_dg(agg, wl_ref[...]) + bl_ref[...]
                     + _dg(h, wr_ref[...]), 0.0)
    xx = jnp.maximum(_dg(agg, gwl_ref[...]) + gbl_ref[...]
                     + _dg(h, gwr_ref[...]), 0.0)
    xa_ref[...] = xa
    _planes(y_ref, jnp.concatenate([xx, xx * xx], axis=1), 4)


def _layer_a(h4, aggs, cnt, wl, bl, wr, gwl, gbl, gwr):
    return pl.pallas_call(
        _layer_a_body,
        grid=(NPAD // BT,),
        in_specs=[_P2, _P2, _RC, _W, _B, _W, _W, _B, _W],
        out_specs=[_R, _P4],
        out_shape=[
            jax.ShapeDtypeStruct((NPAD, F), jnp.float32),
            jax.ShapeDtypeStruct((4, NPAD, PW), jnp.float32),
        ],
    )(h4, aggs, cnt, wl, bl, wr, gwl, gbl, gwr)


def _layer_b_body(h_ref, xa_ref, y_ref, ab_ref, cnt_ref, o_ref):
    cnt = cnt_ref[:, 0:1]
    r = 1.0 / jnp.maximum(cnt, 1.0)
    d = jnp.minimum(cnt, 1.0)
    xx = _cat(y_ref[...])
    ab = ab_ref[...]
    a = jnp.concatenate([ab[i] for i in range(2)], axis=1) * r
    bb = jnp.concatenate([ab[i] for i in range(2, 4)], axis=1) * r
    tau = jnp.tanh((d * xx - 2.0 * a) * xx + bb)
    h = _cat(h_ref[...])
    o = h + tau * (xa_ref[...] - h)
    _planes(o_ref, o, 2)


def _layer_b(h4, xa, y8, ab, cnt):
    return pl.pallas_call(
        _layer_b_body,
        grid=(NPAD // BT,),
        in_specs=[_P2, _R,
                  pl.BlockSpec((2, BT, PW), lambda i: (0, i, 0)), _P4, _RC],
        out_specs=_P2,
        out_shape=_h_struct(),
    )(h4, xa, y8, ab, cnt)


def kernel(x, edge_index, enc_W, enc_b, dec_W, dec_b,
           conv_Wl, conv_bl, conv_Wr, gg_Wl, gg_bl, gg_Wr):
    f32 = jnp.float32
    src = edge_index[0]
    dst = edge_index[1]
    pad = jnp.full((EPAD - E,), N, jnp.int32)
    srcf = jnp.concatenate([src, pad])
    dstf = jnp.concatenate([dst, pad])
    srcp = srcf.reshape(NSUB, J, B)
    dstp = dstf.reshape(NSUB, J, B)
    srcc = srcf.reshape(NSUB, JC, CC)
    dstc = dstf.reshape(NSUB, JC, CC)
    zrows = jnp.zeros((RT, PW), f32)
    zcnt = jnp.zeros((RT, 128), f32)
    ones = jnp.ones((CC, 128), f32)
    xp = jnp.pad(x, ((0, NPAD - N), (0, 0)))

    cnts = _make_counts()(jnp.concatenate([dstc, srcc], axis=0), zcnt, ones)
    cnt_dst = cnts[0]
    cnt_src = cnts[1]
    spmm2 = _make_spmm(2)
    spmm4 = _make_spmm(4)

    blr = conv_bl.reshape(1, F)
    gblr = gg_bl.reshape(1, F)
    h4 = _enc(xp, enc_W, enc_b.reshape(1, F))
    for _ in range(4):
        aggs = spmm2(h4, srcp, dstp, zrows)
        xa, y8 = _layer_a(h4, aggs, cnt_dst, conv_Wl, blr, conv_Wr,
                          gg_Wl, gblr, gg_Wr)
        ab = spmm4(y8, dstp, srcp, zrows)
        h4 = _layer_b(h4, xa, y8, ab, cnt_src)

    dw = jnp.pad(dec_W, ((0, 128 - NCLASS), (0, 0)))
    db = jnp.pad(dec_b, (0, 128 - NCLASS)).reshape(1, 128)
    out = _dec(h4, dw, db)
    return out[:N, :NCLASS]


# speedup vs baseline: 1.6155x; 1.0798x over previous
import functools

import jax
import jax.numpy as jnp
from jax import lax
from jax.experimental import pallas as pl
from jax.experimental.pallas import tpu as pltpu
from jax.experimental.pallas import tpu_sc as plsc

N = 10000
NPAD = 10240
E = 160000
EPAD = 163840
NCLASS = 40
F = 256
PW = 128
NSUB = 16
NCORE = 2
EPS = EPAD // NSUB
B = 128
J = EPS // B
CC = 128
JC = EPS // CC
BT = 640
RT = NPAD // NSUB


def _mesh():
    return plsc.VectorSubcoreMesh(
        core_axis_name="c", subcore_axis_name="s",
        num_cores=NCORE, num_subcores=NSUB)


@functools.lru_cache()
def _make_spmm(npl):
    @functools.partial(
        pl.kernel,
        out_type=jax.ShapeDtypeStruct((npl, NPAD, PW), jnp.float32),
        mesh=_mesh(),
        scratch_types=[
            pltpu.VMEM_SHARED((NPAD, PW), jnp.float32),
            [pltpu.VMEM((B,), jnp.int32) for _ in range(2)],
            [pltpu.VMEM((B,), jnp.int32) for _ in range(4)],
            [pltpu.VMEM((B, PW), jnp.float32) for _ in range(2)],
            [pltpu.SemaphoreType.DMA for _ in range(2)],
            [pltpu.SemaphoreType.DMA for _ in range(2)],
            [pltpu.SemaphoreType.DMA for _ in range(2)],
        ],
    )
    def spmm(data, gidx, sidx, zrows, out,
             acc, gib, sib, dbuf, isems, gsems, ssems):
        c = lax.axis_index("c")
        s = lax.axis_index("s")
        rbase = s * RT
        for p in range(npl // 2):
            q = c + 2 * p
            plane = data.at[q]
            pltpu.sync_copy(zrows, acc.at[pl.ds(rbase, RT)])
            plsc.subcore_barrier()
            pltpu.sync_copy(gidx.at[s, 0], gib[0])
            pltpu.sync_copy(sidx.at[s, 0], sib[0])
            pltpu.async_copy(gidx.at[s, 1], gib[1], isems[1])
            pltpu.async_copy(sidx.at[s, 1], sib[1], isems[1])
            pltpu.async_copy(plane.at[gib[0]], dbuf[0], gsems[0])
            def body(i, _):
                for b in range(4):
                    j = 4 * i + b
                    gb = b % 2
                    gb1 = 1 - gb
                    sb = b
                    sb2 = (b + 2) % 4
                    @pl.when(j + 1 < J)
                    def _():
                        pltpu.make_async_copy(
                            gidx.at[s, j + 1], gib[gb1], isems[gb1]).wait()
                        pltpu.make_async_copy(
                            sidx.at[s, j + 1], sib[(b + 1) % 4],
                            isems[gb1]).wait()
                        @pl.when(j >= 1)
                        def _():
                            pltpu.make_async_copy(
                                dbuf[gb1], acc.at[gib[0]], ssems[gb1]).wait()
                        pltpu.async_copy(
                            plane.at[gib[gb1]], dbuf[gb1], gsems[gb1])
                    pltpu.make_async_copy(
                        plane.at[gib[gb]], dbuf[gb], gsems[gb]).wait()
                    pltpu.async_copy(dbuf[gb], acc.at[sib[sb]], ssems[gb],
                                     add=True)
                    @pl.when(j + 2 < J)
                    def _():
                        pltpu.async_copy(gidx.at[s, j + 2], gib[gb], isems[gb])
                        pltpu.async_copy(sidx.at[s, j + 2], sib[sb2],
                                         isems[gb])
                return 0
            lax.fori_loop(0, J // 4, body, 0)
            pltpu.make_async_copy(dbuf[0], acc.at[gib[0]], ssems[0]).wait()
            pltpu.make_async_copy(dbuf[1], acc.at[gib[1]], ssems[1]).wait()
            plsc.subcore_barrier()
            @pl.when(c == 0)
            def _():
                pltpu.sync_copy(acc.at[pl.ds(rbase, RT)],
                                out.at[2 * p, pl.ds(rbase, RT)])
            @pl.when(c == 1)
            def _():
                pltpu.sync_copy(acc.at[pl.ds(rbase, RT)],
                                out.at[2 * p + 1, pl.ds(rbase, RT)])
            if p + 1 < npl // 2:
                plsc.subcore_barrier()

    return spmm


@functools.lru_cache()
def _make_counts():
    @functools.partial(
        pl.kernel,
        out_type=jax.ShapeDtypeStruct((NCORE, NPAD, 128), jnp.float32),
        mesh=_mesh(),
        scratch_types=[
            pltpu.VMEM_SHARED((NPAD, 128), jnp.float32),
            pltpu.VMEM((JC, CC), jnp.int32),
            pltpu.VMEM((CC, 128), jnp.float32),
            pltpu.SemaphoreType.DMA,
        ],
    )
    def counts(eidx, zcnt, ones, out, acc, ii, ob, csem):
        c = lax.axis_index("c")
        s = lax.axis_index("s")
        rbase = s * RT
        @pl.when(c == 0)
        def _():
            pltpu.sync_copy(eidx.at[s], ii)
        @pl.when(c == 1)
        def _():
            pltpu.sync_copy(eidx.at[NSUB + s], ii)
        pltpu.sync_copy(ones, ob)
        pltpu.sync_copy(zcnt, acc.at[pl.ds(rbase, RT)])
        plsc.subcore_barrier()
        def body(i, _):
            for b in range(4):
                pltpu.async_copy(ob, acc.at[ii.at[4 * i + b]], csem, add=True)
            for b in range(4):
                pltpu.make_async_copy(ob, acc.at[ii.at[4 * i]], csem).wait()
            return 0
        lax.fori_loop(0, JC // 4, body, 0)
        plsc.subcore_barrier()
        @pl.when(c == 0)
        def _():
            pltpu.sync_copy(acc.at[pl.ds(rbase, RT)],
                            out.at[0, pl.ds(rbase, RT)])
        @pl.when(c == 1)
        def _():
            pltpu.sync_copy(acc.at[pl.ds(rbase, RT)],
                            out.at[1, pl.ds(rbase, RT)])

    return counts


def _dg(a, w):
    return lax.dot_general(a, w, (((1,), (1,)), ((), ())),
                           preferred_element_type=jnp.float32)


def _cat(pref):
    return jnp.concatenate([pref[i] for i in range(pref.shape[0])], axis=1)


def _planes(o_ref, y, npl):
    for qq in range(npl):
        o_ref[qq] = y[:, qq * PW:(qq + 1) * PW]


_P2 = pl.BlockSpec((2, BT, PW), lambda i: (0, i, 0))
_P4 = pl.BlockSpec((4, BT, PW), lambda i: (0, i, 0))
_R = pl.BlockSpec((BT, F), lambda i: (i, 0))
_RC = pl.BlockSpec((BT, 128), lambda i: (i, 0))
_W = pl.BlockSpec((F, F), lambda i: (0, 0))
_B = pl.BlockSpec((1, F), lambda i: (0, 0))


def _h_struct():
    return jax.ShapeDtypeStruct((2, NPAD, PW), jnp.float32)


def _enc_body(x_ref, w_ref, b_ref, o_ref):
    y = jnp.maximum(_dg(x_ref[...], w_ref[...]) + b_ref[...], 0.0)
    _planes(o_ref, y, 2)


def _enc(x, w, b):
    return pl.pallas_call(
        _enc_body,
        grid=(NPAD // BT,),
        in_specs=[_R, _W, _B],
        out_specs=_P2,
        out_shape=_h_struct(),
    )(x, w, b)


def _dec_body(h_ref, w_ref, b_ref, o_ref):
    o_ref[...] = _dg(_cat(h_ref[...]), w_ref[...]) + b_ref[...]


def _dec(h4, w, b):
    return pl.pallas_call(
        _dec_body,
        grid=(NPAD // BT,),
        in_specs=[
            _P2,
            pl.BlockSpec((128, F), lambda i: (0, 0)),
            pl.BlockSpec((1, 128), lambda i: (0, 0)),
        ],
        out_specs=_RC,
        out_shape=jax.ShapeDtypeStruct((NPAD, 128), jnp.float32),
    )(h4, w, b)


def _layer_a_body(h_ref, agg_ref, cnt_ref, wl_ref, bl_ref, wr_ref,
                  gwl_ref, gbl_ref, gwr_ref, xa_ref, y_ref):
    cnt = cnt_ref[:, 0:1]
    r = 1.0 / jnp.maximum(cnt, 1.0)
    agg = _cat(agg_ref[...]) * r
    h = _cat(h_ref[...])
    xa = jnp.maximum(_dg(agg, wl_ref[...]) + bl_ref[...]
                     + _dg(h, wr_ref[...]), 0.0)
    xx = jnp.maximum(_dg(agg, gwl_ref[...]) + gbl_ref[...]
                     + _dg(h, gwr_ref[...]), 0.0)
    xa_ref[...] = xa
    _planes(y_ref, jnp.concatenate([xx, xx * xx], axis=1), 4)


def _layer_a(h4, aggs, cnt, wl, bl, wr, gwl, gbl, gwr):
    return pl.pallas_call(
        _layer_a_body,
        grid=(NPAD // BT,),
        in_specs=[_P2, _P2, _RC, _W, _B, _W, _W, _B, _W],
        out_specs=[_R, _P4],
        out_shape=[
            jax.ShapeDtypeStruct((NPAD, F), jnp.float32),
            jax.ShapeDtypeStruct((4, NPAD, PW), jnp.float32),
        ],
    )(h4, aggs, cnt, wl, bl, wr, gwl, gbl, gwr)


def _layer_b_body(h_ref, xa_ref, y_ref, ab_ref, cnt_ref, o_ref):
    cnt = cnt_ref[:, 0:1]
    r = 1.0 / jnp.maximum(cnt, 1.0)
    d = jnp.minimum(cnt, 1.0)
    xx = _cat(y_ref[...])
    ab = ab_ref[...]
    a = jnp.concatenate([ab[i] for i in range(2)], axis=1) * r
    bb = jnp.concatenate([ab[i] for i in range(2, 4)], axis=1) * r
    tau = jnp.tanh((d * xx - 2.0 * a) * xx + bb)
    h = _cat(h_ref[...])
    o = h + tau * (xa_ref[...] - h)
    _planes(o_ref, o, 2)


def _layer_b(h4, xa, y8, ab, cnt):
    return pl.pallas_call(
        _layer_b_body,
        grid=(NPAD // BT,),
        in_specs=[_P2, _R,
                  pl.BlockSpec((2, BT, PW), lambda i: (0, i, 0)), _P4, _RC],
        out_specs=_P2,
        out_shape=_h_struct(),
    )(h4, xa, y8, ab, cnt)


def kernel(x, edge_index, enc_W, enc_b, dec_W, dec_b,
           conv_Wl, conv_bl, conv_Wr, gg_Wl, gg_bl, gg_Wr):
    f32 = jnp.float32
    src = edge_index[0]
    dst = edge_index[1]
    pad = jnp.full((EPAD - E,), N, jnp.int32)
    srcf = jnp.concatenate([src, pad])
    dstf = jnp.concatenate([dst, pad])
    srcp = srcf.reshape(NSUB, J, B)
    dstp = dstf.reshape(NSUB, J, B)
    srcc = srcf.reshape(NSUB, JC, CC)
    dstc = dstf.reshape(NSUB, JC, CC)
    zrows = jnp.zeros((RT, PW), f32)
    zcnt = jnp.zeros((RT, 128), f32)
    ones = jnp.ones((CC, 128), f32)
    xp = jnp.pad(x, ((0, NPAD - N), (0, 0)))

    cnts = _make_counts()(jnp.concatenate([dstc, srcc], axis=0), zcnt, ones)
    cnt_dst = cnts[0]
    cnt_src = cnts[1]
    spmm2 = _make_spmm(2)
    spmm4 = _make_spmm(4)

    blr = conv_bl.reshape(1, F)
    gblr = gg_bl.reshape(1, F)
    h4 = _enc(xp, enc_W, enc_b.reshape(1, F))
    for _ in range(4):
        aggs = spmm2(h4, srcp, dstp, zrows)
        xa, y8 = _layer_a(h4, aggs, cnt_dst, conv_Wl, blr, conv_Wr,
                          gg_Wl, gblr, gg_Wr)
        ab = spmm4(y8, dstp, srcp, zrows)
        h4 = _layer_b(h4, xa, y8, ab, cnt_src)

    dw = jnp.pad(dec_W, ((0, 128 - NCLASS), (0, 0)))
    db = jnp.pad(dec_b, (0, 128 - NCLASS)).reshape(1, 128)
    out = _dec(h4, dw, db)
    return out[:N, :NCLASS]
